# Initial kernel scaffold; baseline (speedup 1.0000x reference)
import functools
import jax
import jax.numpy as jnp
from jax import lax
from jax.experimental import pallas as pl
from jax.experimental.pallas import tpu as pltpu
from jax.experimental.pallas import tpu_sc as plsc

N_NODES = 10000
N_EDGES = 320000
D_FEAT = 128
KU = 64
W = 80

NC = 2
NS = 16
NW = NC * NS
E_PER_W = N_EDGES // NW
CHUNK = 80
N_IT = E_PER_W // CHUNK
NBUF = 5
N_GROUP = N_IT // NBUF
ROWS_PER_TILE = N_NODES // NS
ZROWS = 125


def _scatter_mean_sc(y_aug, row_g, col_g):
  mesh = plsc.VectorSubcoreMesh(core_axis_name="c", subcore_axis_name="s")

  @functools.partial(
      pl.kernel,
      out_type=jax.ShapeDtypeStruct((NC, N_NODES, W), jnp.float32),
      mesh=mesh,
      scratch_types=[
          pltpu.VMEM((N_IT, CHUNK), jnp.int32),
          pltpu.VMEM((N_IT, CHUNK), jnp.int32),
          pltpu.VMEM((NBUF, CHUNK, W), jnp.float32),
          pltpu.VMEM((ZROWS, W), jnp.float32),
          pltpu.VMEM_SHARED((N_NODES, W), jnp.float32),
          pltpu.SemaphoreType.DMA,
          pltpu.SemaphoreType.DMA,
      ],
  )
  def k(y_hbm, row_hbm, col_hbm, out_hbm, colv, rowv, bufs, zbuf, acc, gsem, ssem):
    cid = lax.axis_index("c")
    sid = lax.axis_index("s")
    wid = sid * NC + cid

    def zbody(i, carry):
      for j in range(W // 16):
        zbuf[i, pl.ds(j * 16, 16)] = jnp.zeros((16,), jnp.float32)
      return carry
    lax.fori_loop(0, ZROWS, zbody, 0)
    for kk in range(ROWS_PER_TILE // ZROWS):
      pltpu.sync_copy(
          zbuf, acc.at[pl.ds(sid * ROWS_PER_TILE + kk * ZROWS, ZROWS)])
    plsc.subcore_barrier()

    pltpu.sync_copy(col_hbm.at[wid], colv)
    pltpu.sync_copy(row_hbm.at[wid], rowv)

    def outer(g, carry):
      base = g * NBUF
      gds = [
          pltpu.async_copy(y_hbm.at[colv.at[base + b]], bufs.at[b], gsem)
          for b in range(NBUF)
      ]
      sds = []
      for b in range(NBUF):
        gds[b].wait()
        sds.append(
            pltpu.async_copy(bufs.at[b], acc.at[rowv.at[base + b]], ssem,
                             add=True))
      for d in sds:
        d.wait()
      return carry
    lax.fori_loop(0, N_GROUP, outer, 0)

    plsc.subcore_barrier()
    pltpu.sync_copy(
        acc.at[pl.ds(sid * ROWS_PER_TILE, ROWS_PER_TILE)],
        out_hbm.at[cid, pl.ds(sid * ROWS_PER_TILE, ROWS_PER_TILE)])

  return k(y_aug, row_g, col_g)


def _project_tc(x, nk_pad):
  def body(x_ref, nk_ref, out_ref):
    y = jnp.dot(x_ref[...], nk_ref[...], preferred_element_type=jnp.float32)
    col = lax.broadcasted_iota(jnp.int32, (N_NODES, W), 1)
    out_ref[...] = y + jnp.where(col == KU, 1.0, 0.0)

  return pl.pallas_call(
      body,
      out_shape=jax.ShapeDtypeStruct((N_NODES, W), jnp.float32),
  )(x, nk_pad)


def _combine_tc(x, sk, parts, bias2d):
  def body(x_ref, sk_ref, p_ref, b_ref, out_ref):
    p = p_ref[0] + p_ref[1]
    counts = jnp.maximum(p[:, KU:KU + 1], 1.0)
    mean = p[:, :KU] / counts
    self_msg = jnp.dot(x_ref[...], sk_ref[...],
                       preferred_element_type=jnp.float32)
    h = jnp.concatenate([self_msg, mean], axis=1) + b_ref[...]
    out_ref[...] = jnp.maximum(h, 0.0)

  return pl.pallas_call(
      body,
      out_shape=jax.ShapeDtypeStruct((N_NODES, 2 * KU), jnp.float32),
  )(x, sk, parts, bias2d)


def kernel(x, edge_index, self_kernel, neighbor_kernel, bias):
  nk_pad = jnp.zeros((D_FEAT, W), jnp.float32).at[:, :KU].set(neighbor_kernel)
  y_aug = _project_tc(x, nk_pad)
  row_g = edge_index[0].reshape(NW, N_IT, CHUNK)
  col_g = edge_index[1].reshape(NW, N_IT, CHUNK)
  parts = _scatter_mean_sc(y_aug, row_g, col_g)
  return _combine_tc(x, self_kernel, parts, bias.reshape(1, 2 * KU))


# speedup vs baseline: 3.7431x; 3.7431x over previous
import functools
import jax
import jax.numpy as jnp
from jax import lax
from jax.experimental import pallas as pl
from jax.experimental.pallas import tpu as pltpu
from jax.experimental.pallas import tpu_sc as plsc

N_NODES = 10000
N_EDGES = 320000
D_FEAT = 128
KU = 64
W = 128

NC = 1
NS = 16
NW = NC * NS
CHUNK = 128
BLK = 16
N_BLOCKS = 10
N_IT = N_BLOCKS * BLK
E_PER_W = N_IT * CHUNK
N_EDGES_PAD = NW * E_PER_W
NBUF = 2
N_PAD = 10240
ROWS_PER_TILE = N_PAD // NS
PAD_ROW = N_PAD - 8


def _scatter_mean_sc(y_aug, row_g, col_g):
  mesh = plsc.VectorSubcoreMesh(core_axis_name="c", subcore_axis_name="s",
                                num_cores=NC)

  @functools.partial(
      pl.kernel,
      out_type=pltpu.MemorySpace.HBM((NC, NS, ROWS_PER_TILE, W), jnp.float32),
      mesh=mesh,
      scratch_types=[
          pltpu.VMEM((BLK, CHUNK), jnp.int32),
          pltpu.VMEM((BLK, CHUNK), jnp.int32),
          pltpu.VMEM((NBUF, CHUNK, W), jnp.float32),
          pltpu.VMEM_SHARED((N_PAD, W), jnp.float32),
          pltpu.SemaphoreType.DMA,
          pltpu.SemaphoreType.DMA,
      ],
  )
  def k(y_hbm, row_hbm, col_hbm, out_hbm, colb, rowb, bufs, acc, gsem, ssem):
    cid = lax.axis_index("c")
    sid = lax.axis_index("s")
    wid = sid * NC + cid

    def zbody(i, carry):
      for j in range(W // 16):
        bufs[0, i, pl.ds(j * 16, 16)] = jnp.zeros((16,), jnp.float32)
      return carry
    lax.fori_loop(0, CHUNK, zbody, 0)
    for kk in range(ROWS_PER_TILE // CHUNK):
      pltpu.sync_copy(
          bufs.at[0], acc.at[pl.ds(sid * ROWS_PER_TILE + kk * CHUNK, CHUNK)])
    plsc.subcore_barrier()

    def outer(go, carry):
      pltpu.sync_copy(col_hbm.at[wid, go], colb)
      pltpu.sync_copy(row_hbm.at[wid, go], rowb)
      for pair in range(BLK // NBUF):
        c0 = NBUF * pair
        g0 = pltpu.async_copy(y_hbm.at[colb.at[c0]], bufs.at[0], gsem)
        g1 = pltpu.async_copy(y_hbm.at[colb.at[c0 + 1]], bufs.at[1], gsem)
        g0.wait()
        s0 = pltpu.async_copy(bufs.at[0], acc.at[rowb.at[c0]], ssem, add=True)
        g1.wait()
        s1 = pltpu.async_copy(bufs.at[1], acc.at[rowb.at[c0 + 1]], ssem,
                              add=True)
        s0.wait()
        s1.wait()
      return carry
    lax.fori_loop(0, N_BLOCKS, outer, 0)

    plsc.subcore_barrier()
    pltpu.sync_copy(
        acc.at[pl.ds(sid * ROWS_PER_TILE, ROWS_PER_TILE)],
        out_hbm.at[cid, sid])

  return k(y_aug, row_g, col_g)


def _project_tc(x, nk_pad):
  def body(x_ref, nk_ref, out_ref):
    y = jnp.dot(x_ref[...], nk_ref[...], preferred_element_type=jnp.float32)
    col = lax.broadcasted_iota(jnp.int32, (N_NODES, W), 1)
    out_ref[...] = y + jnp.where(col == KU, 1.0, 0.0)

  return pl.pallas_call(
      body,
      out_shape=jax.ShapeDtypeStruct((N_NODES, W), jnp.float32),
  )(x, nk_pad)


def _combine_tc(x, sk, parts, bias2d):
  def body(x_ref, sk_ref, p_ref, b_ref, out_ref):
    p = p_ref[0]
    for c in range(1, NC):
      p = p + p_ref[c]
    p = p[:N_NODES]
    counts = jnp.maximum(p[:, KU:KU + 1], 1.0)
    mean = p[:, :KU] / counts
    self_msg = jnp.dot(x_ref[...], sk_ref[...],
                       preferred_element_type=jnp.float32)
    h = jnp.concatenate([self_msg, mean], axis=1) + b_ref[...]
    out_ref[...] = jnp.maximum(h, 0.0)

  return pl.pallas_call(
      body,
      out_shape=jax.ShapeDtypeStruct((N_NODES, 2 * KU), jnp.float32),
  )(x, sk, parts, bias2d)


def kernel(x, edge_index, self_kernel, neighbor_kernel, bias):
  nk_pad = jnp.zeros((D_FEAT, W), jnp.float32).at[:, :KU].set(neighbor_kernel)
  y_aug = _project_tc(x, nk_pad)
  pad = N_EDGES_PAD - N_EDGES
  row_p = jnp.concatenate(
      [edge_index[0], jnp.full((pad,), PAD_ROW, jnp.int32)])
  col_p = jnp.concatenate([edge_index[1], jnp.zeros((pad,), jnp.int32)])
  row_g = row_p.reshape(NW, N_BLOCKS, BLK, CHUNK)
  col_g = col_p.reshape(NW, N_BLOCKS, BLK, CHUNK)
  parts = _scatter_mean_sc(y_aug, row_g, col_g).reshape(NC, N_PAD, W)
  return _combine_tc(x, self_kernel, parts, bias.reshape(1, 2 * KU))


# speedup vs baseline: 4.0839x; 1.0910x over previous
import functools
import jax
import jax.numpy as jnp
from jax import lax
from jax.experimental import pallas as pl
from jax.experimental.pallas import tpu as pltpu
from jax.experimental.pallas import tpu_sc as plsc

N_NODES = 10000
N_EDGES = 320000
D_FEAT = 128
KU = 64
W = 128

NC = 1
NS = 16
NW = NC * NS
CHUNK = 128
BLK = 16
N_BLOCKS = 10
N_IT = N_BLOCKS * BLK
E_PER_W = N_IT * CHUNK
N_EDGES_PAD = NW * E_PER_W
NBUF = 2
N_PAD = 10240
ROWS_PER_TILE = N_PAD // NS
PAD_ROW = N_PAD - 8


def _scatter_mean_sc(y_aug, row_g, col_g):
  mesh = plsc.VectorSubcoreMesh(core_axis_name="c", subcore_axis_name="s",
                                num_cores=NC)

  @functools.partial(
      pl.kernel,
      out_type=pltpu.MemorySpace.HBM((NC, NS, ROWS_PER_TILE, W), jnp.float32),
      mesh=mesh,
      scratch_types=[
          pltpu.VMEM((BLK, CHUNK), jnp.int32),
          pltpu.VMEM((BLK, CHUNK), jnp.int32),
          pltpu.VMEM((NBUF, CHUNK, W), jnp.float32),
          pltpu.VMEM_SHARED((N_PAD, W), jnp.float32),
          pltpu.SemaphoreType.DMA,
          pltpu.SemaphoreType.DMA,
      ],
  )
  def k(y_hbm, row_hbm, col_hbm, out_hbm, colb, rowb, bufs, acc, gsem, ssem):
    cid = lax.axis_index("c")
    sid = lax.axis_index("s")
    wid = sid * NC + cid

    def zbody(i, carry):
      for j in range(W // 16):
        bufs[0, i, pl.ds(j * 16, 16)] = jnp.zeros((16,), jnp.float32)
      return carry
    lax.fori_loop(0, CHUNK, zbody, 0)
    for kk in range(ROWS_PER_TILE // CHUNK):
      pltpu.sync_copy(
          bufs.at[0], acc.at[pl.ds(sid * ROWS_PER_TILE + kk * CHUNK, CHUNK)])
    plsc.subcore_barrier()

    def outer(go, carry):
      pltpu.sync_copy(col_hbm.at[wid, go], colb)
      pltpu.sync_copy(row_hbm.at[wid, go], rowb)
      gd = [None] * BLK
      sd = [None] * BLK
      gd[0] = pltpu.async_copy(y_hbm.at[colb.at[0]], bufs.at[0], gsem)
      for c in range(BLK):
        if c >= 1:
          sd[c - 1].wait()
        if c + 1 < BLK:
          gd[c + 1] = pltpu.async_copy(
              y_hbm.at[colb.at[c + 1]], bufs.at[(c + 1) % NBUF], gsem)
        gd[c].wait()
        sd[c] = pltpu.async_copy(bufs.at[c % NBUF], acc.at[rowb.at[c]], ssem,
                                 add=True)
      sd[BLK - 1].wait()
      return carry
    lax.fori_loop(0, N_BLOCKS, outer, 0)

    plsc.subcore_barrier()
    pltpu.sync_copy(
        acc.at[pl.ds(sid * ROWS_PER_TILE, ROWS_PER_TILE)],
        out_hbm.at[cid, sid])

  return k(y_aug, row_g, col_g)


def _project_tc(x, nk_pad):
  def body(x_ref, nk_ref, out_ref):
    y = jnp.dot(x_ref[...], nk_ref[...], preferred_element_type=jnp.float32)
    col = lax.broadcasted_iota(jnp.int32, (N_NODES, W), 1)
    out_ref[...] = y + jnp.where(col == KU, 1.0, 0.0)

  return pl.pallas_call(
      body,
      out_shape=jax.ShapeDtypeStruct((N_NODES, W), jnp.float32),
  )(x, nk_pad)


def _combine_tc(x, sk, parts, bias2d):
  def body(x_ref, sk_ref, p_ref, b_ref, out_ref):
    p = p_ref[0]
    for c in range(1, NC):
      p = p + p_ref[c]
    p = p[:N_NODES]
    counts = jnp.maximum(p[:, KU:KU + 1], 1.0)
    mean = p[:, :KU] / counts
    self_msg = jnp.dot(x_ref[...], sk_ref[...],
                       preferred_element_type=jnp.float32)
    h = jnp.concatenate([self_msg, mean], axis=1) + b_ref[...]
    out_ref[...] = jnp.maximum(h, 0.0)

  return pl.pallas_call(
      body,
      out_shape=jax.ShapeDtypeStruct((N_NODES, 2 * KU), jnp.float32),
  )(x, sk, parts, bias2d)


def kernel(x, edge_index, self_kernel, neighbor_kernel, bias):
  nk_pad = jnp.zeros((D_FEAT, W), jnp.float32).at[:, :KU].set(neighbor_kernel)
  y_aug = _project_tc(x, nk_pad)
  pad = N_EDGES_PAD - N_EDGES
  row_p = jnp.concatenate(
      [edge_index[0], jnp.full((pad,), PAD_ROW, jnp.int32)])
  col_p = jnp.concatenate([edge_index[1], jnp.zeros((pad,), jnp.int32)])
  row_g = row_p.reshape(NW, N_BLOCKS, BLK, CHUNK)
  col_g = col_p.reshape(NW, N_BLOCKS, BLK, CHUNK)
  parts = _scatter_mean_sc(y_aug, row_g, col_g).reshape(NC, N_PAD, W)
  return _combine_tc(x, self_kernel, parts, bias.reshape(1, 2 * KU))


# speedup vs baseline: 4.3458x; 1.0641x over previous
import functools
import jax
import jax.numpy as jnp
from jax import lax
from jax.experimental import pallas as pl
from jax.experimental.pallas import tpu as pltpu
from jax.experimental.pallas import tpu_sc as plsc

N_NODES = 10000
N_EDGES = 320000
D_FEAT = 128
KU = 64
W = 128

NC = 2
NS = 16
NW = NC * NS
CHUNK = 128
BLK = 16
N_BLOCKS = 5
N_IT = N_BLOCKS * BLK
E_PER_W = N_IT * CHUNK
N_EDGES_PAD = NW * E_PER_W
NBUF = 2
N_PAD = 10240
ROWS_PER_TILE = N_PAD // NS
PAD_ROW = N_PAD - 8


def _scatter_mean_sc(y_aug, row_g, col_g):
  mesh = plsc.VectorSubcoreMesh(core_axis_name="c", subcore_axis_name="s",
                                num_cores=NC)

  @functools.partial(
      pl.kernel,
      out_type=pltpu.MemorySpace.HBM((NC, NS, ROWS_PER_TILE, W), jnp.float32),
      mesh=mesh,
      scratch_types=[
          pltpu.VMEM((BLK, CHUNK), jnp.int32),
          pltpu.VMEM((BLK, CHUNK), jnp.int32),
          pltpu.VMEM((NBUF, CHUNK, W), jnp.float32),
          pltpu.VMEM_SHARED((N_PAD, W), jnp.float32),
          pltpu.SemaphoreType.DMA,
          pltpu.SemaphoreType.DMA,
      ],
  )
  def k(y_hbm, row_hbm, col_hbm, out_hbm, colb, rowb, bufs, acc, gsem, ssem):
    cid = lax.axis_index("c")
    sid = lax.axis_index("s")
    wid = sid * NC + cid

    def zbody(i, carry):
      for j in range(W // 16):
        bufs[0, i, pl.ds(j * 16, 16)] = jnp.zeros((16,), jnp.float32)
      return carry
    lax.fori_loop(0, CHUNK, zbody, 0)
    for kk in range(ROWS_PER_TILE // CHUNK):
      pltpu.sync_copy(
          bufs.at[0], acc.at[pl.ds(sid * ROWS_PER_TILE + kk * CHUNK, CHUNK)])
    plsc.subcore_barrier()

    def outer(go, carry):
      pltpu.sync_copy(col_hbm.at[wid, go], colb)
      pltpu.sync_copy(row_hbm.at[wid, go], rowb)
      gd = [None] * BLK
      sd = [None] * BLK
      gd[0] = pltpu.async_copy(y_hbm.at[colb.at[0]], bufs.at[0], gsem)
      for c in range(BLK):
        if c >= 1:
          sd[c - 1].wait()
        if c + 1 < BLK:
          gd[c + 1] = pltpu.async_copy(
              y_hbm.at[colb.at[c + 1]], bufs.at[(c + 1) % NBUF], gsem)
        gd[c].wait()
        sd[c] = pltpu.async_copy(bufs.at[c % NBUF], acc.at[rowb.at[c]], ssem,
                                 add=True)
      sd[BLK - 1].wait()
      return carry
    lax.fori_loop(0, N_BLOCKS, outer, 0)

    plsc.subcore_barrier()
    pltpu.sync_copy(
        acc.at[pl.ds(sid * ROWS_PER_TILE, ROWS_PER_TILE)],
        out_hbm.at[cid, sid])

  return k(y_aug, row_g, col_g)


def _project_tc(x, nk_pad):
  def body(x_ref, nk_ref, out_ref):
    y = jnp.dot(x_ref[...], nk_ref[...], preferred_element_type=jnp.float32)
    col = lax.broadcasted_iota(jnp.int32, (N_NODES, W), 1)
    out_ref[...] = y + jnp.where(col == KU, 1.0, 0.0)

  return pl.pallas_call(
      body,
      out_shape=jax.ShapeDtypeStruct((N_NODES, W), jnp.float32),
  )(x, nk_pad)


def _combine_tc(x, sk, parts, bias2d):
  def body(x_ref, sk_ref, p_ref, b_ref, out_ref):
    p = p_ref[0]
    for c in range(1, NC):
      p = p + p_ref[c]
    p = p[:N_NODES]
    counts = jnp.maximum(p[:, KU:KU + 1], 1.0)
    mean = p[:, :KU] / counts
    self_msg = jnp.dot(x_ref[...], sk_ref[...],
                       preferred_element_type=jnp.float32)
    h = jnp.concatenate([self_msg, mean], axis=1) + b_ref[...]
    out_ref[...] = jnp.maximum(h, 0.0)

  return pl.pallas_call(
      body,
      out_shape=jax.ShapeDtypeStruct((N_NODES, 2 * KU), jnp.float32),
  )(x, sk, parts, bias2d)


def kernel(x, edge_index, self_kernel, neighbor_kernel, bias):
  nk_pad = jnp.zeros((D_FEAT, W), jnp.float32).at[:, :KU].set(neighbor_kernel)
  y_aug = _project_tc(x, nk_pad)
  pad = N_EDGES_PAD - N_EDGES
  row_p = jnp.concatenate(
      [edge_index[0], jnp.full((pad,), PAD_ROW, jnp.int32)])
  col_p = jnp.concatenate([edge_index[1], jnp.zeros((pad,), jnp.int32)])
  row_g = row_p.reshape(NW, N_BLOCKS, BLK, CHUNK)
  col_g = col_p.reshape(NW, N_BLOCKS, BLK, CHUNK)
  parts = _scatter_mean_sc(y_aug, row_g, col_g).reshape(NC, N_PAD, W)
  return _combine_tc(x, self_kernel, parts, bias.reshape(1, 2 * KU))
